# Initial kernel scaffold; baseline (speedup 1.0000x reference)
import functools

import jax
import jax.numpy as jnp
from jax import lax
from jax.experimental import pallas as pl
from jax.experimental.pallas import tpu as pltpu
from jax.experimental.pallas import tpu_sc as plsc

_NC = 2
_NS = 16
_NW = _NC * _NS


def _silu(x):
    return x * jax.nn.sigmoid(x)


def _proj_body(h_ref, w1at_ref, w1bt_ref, s_ref, t_ref):
    hb = h_ref[...]
    s_ref[...] = jnp.dot(hb, w1at_ref[...], preferred_element_type=jnp.float32)
    t_ref[...] = jnp.dot(hb, w1bt_ref[...], preferred_element_type=jnp.float32)


def _sc_gather(hp_s, hp_t, row, col, *, chunk):
    e = row.shape[0]
    d = hp_s.shape[1]
    epw = e // _NW
    mesh = plsc.VectorSubcoreMesh(core_axis_name="c", subcore_axis_name="s")

    @functools.partial(
        pl.kernel,
        out_type=[jax.ShapeDtypeStruct((e, d), jnp.float32),
                  jax.ShapeDtypeStruct((e, d), jnp.float32)],
        mesh=mesh,
        scratch_types=[pltpu.VMEM((chunk,), jnp.int32),
                       pltpu.VMEM((chunk, d), jnp.float32),
                       pltpu.SemaphoreType.DMA],
    )
    def k(hp_s_hbm, hp_t_hbm, row_hbm, col_hbm, s_hbm, t_hbm, idx_v, rows_v, sem):
        wid = lax.axis_index("s") * _NC + lax.axis_index("c")
        base = wid * epw

        @pl.loop(0, epw, step=chunk)
        def _(k0):
            off = base + k0
            pltpu.sync_copy(row_hbm.at[pl.ds(off, chunk)], idx_v)
            pltpu.async_copy(hp_s_hbm.at[idx_v], rows_v, sem).wait()
            pltpu.sync_copy(rows_v, s_hbm.at[pl.ds(off, chunk)])
            pltpu.sync_copy(col_hbm.at[pl.ds(off, chunk)], idx_v)
            pltpu.async_copy(hp_t_hbm.at[idx_v], rows_v, sem).wait()
            pltpu.sync_copy(rows_v, t_hbm.at[pl.ds(off, chunk)])

    return k(hp_s, hp_t, row, col)


def _edge_body(s_ref, t_ref, cd_ref, w1c_ref, b1_ref, w2t_ref, b2_ref, m_ref):
    cd = cd_ref[...]
    radial = jnp.sum(cd * cd, axis=1, keepdims=True)
    x = s_ref[...] + t_ref[...] + radial * w1c_ref[...] + b1_ref[...]
    x = _silu(x)
    y = jnp.dot(x, w2t_ref[...], preferred_element_type=jnp.float32) + b2_ref[...]
    m_ref[...] = _silu(y)


def _sc_scatter(m, row, zeros, n, *, chunk):
    e, d = m.shape
    epw = e // _NW
    rps = n // _NS
    mesh = plsc.VectorSubcoreMesh(core_axis_name="c", subcore_axis_name="s")

    @functools.partial(
        pl.kernel,
        out_type=jax.ShapeDtypeStruct((_NC, n, d), jnp.float32),
        mesh=mesh,
        scratch_types=[pltpu.VMEM((chunk,), jnp.int32),
                       pltpu.VMEM((chunk, d), jnp.float32),
                       pltpu.VMEM_SHARED((n, d), jnp.float32),
                       pltpu.SemaphoreType.DMA],
    )
    def k(m_hbm, row_hbm, z_hbm, out_hbm, idx_v, m_v, agg_sh, sem):
        cid = lax.axis_index("c")
        sid = lax.axis_index("s")
        wid = sid * _NC + cid
        pltpu.sync_copy(z_hbm.at[pl.ds(sid * rps, rps)],
                        agg_sh.at[pl.ds(sid * rps, rps)])
        plsc.subcore_barrier()
        base = wid * epw

        @pl.loop(0, epw, step=chunk)
        def _(k0):
            off = base + k0
            pltpu.sync_copy(row_hbm.at[pl.ds(off, chunk)], idx_v)
            pltpu.sync_copy(m_hbm.at[pl.ds(off, chunk)], m_v)
            pltpu.sync_copy(m_v, agg_sh.at[idx_v], add=True)

        plsc.subcore_barrier()
        pltpu.sync_copy(agg_sh.at[pl.ds(sid * rps, rps)],
                        out_hbm.at[cid].at[pl.ds(sid * rps, rps)])

    return k(m, row, zeros)


def _node_body(h_ref, agg_ref, w3at_ref, w3bt_ref, b3_ref, w4t_ref, b4_ref, o_ref):
    agg = agg_ref[0] + agg_ref[1]
    x = (jnp.dot(h_ref[...], w3at_ref[...], preferred_element_type=jnp.float32)
         + jnp.dot(agg, w3bt_ref[...], preferred_element_type=jnp.float32)
         + b3_ref[...])
    x = _silu(x)
    o_ref[...] = jnp.dot(x, w4t_ref[...], preferred_element_type=jnp.float32) + b4_ref[...]


def kernel(h, edges_index, coord_diff, w1, b1, w2, b2, w3, b3, w4, b4):
    n, d = h.shape
    e = edges_index.shape[1]
    hd = w1.shape[0]

    ei = edges_index.astype(jnp.int32)
    row = ei[0]
    col = ei[1]

    w1at = w1[:, :d].T
    w1bt = w1[:, d:2 * d].T
    w1c = w1[:, 2 * d].reshape(1, hd)
    b1r = b1.reshape(1, hd)
    w2t = w2.T
    b2r = b2.reshape(1, hd)
    w3at = w3[:, :d].T
    w3bt = w3[:, d:].T
    b3r = b3.reshape(1, hd)
    w4t = w4.T
    b4r = b4.reshape(1, d)

    nblk = 2000
    hp_s, hp_t = pl.pallas_call(
        _proj_body,
        grid=(n // nblk,),
        in_specs=[pl.BlockSpec((nblk, d), lambda i: (i, 0)),
                  pl.BlockSpec((d, hd), lambda i: (0, 0)),
                  pl.BlockSpec((d, hd), lambda i: (0, 0))],
        out_specs=[pl.BlockSpec((nblk, hd), lambda i: (i, 0)),
                   pl.BlockSpec((nblk, hd), lambda i: (i, 0))],
        out_shape=[jax.ShapeDtypeStruct((n, hd), jnp.float32),
                   jax.ShapeDtypeStruct((n, hd), jnp.float32)],
    )(h, w1at, w1bt)

    s_g, t_g = _sc_gather(hp_s, hp_t, row, col, chunk=400)

    eblk = 2000
    m = pl.pallas_call(
        _edge_body,
        grid=(e // eblk,),
        in_specs=[pl.BlockSpec((eblk, hd), lambda i: (i, 0)),
                  pl.BlockSpec((eblk, hd), lambda i: (i, 0)),
                  pl.BlockSpec((eblk, 3), lambda i: (i, 0)),
                  pl.BlockSpec((1, hd), lambda i: (0, 0)),
                  pl.BlockSpec((1, hd), lambda i: (0, 0)),
                  pl.BlockSpec((hd, hd), lambda i: (0, 0)),
                  pl.BlockSpec((1, hd), lambda i: (0, 0))],
        out_specs=pl.BlockSpec((eblk, hd), lambda i: (i, 0)),
        out_shape=jax.ShapeDtypeStruct((e, hd), jnp.float32),
    )(s_g, t_g, coord_diff, w1c, b1r, w2t, b2r)

    zeros = jnp.zeros((n, hd), jnp.float32)
    agg2 = _sc_scatter(m, row, zeros, n, chunk=400)

    out = pl.pallas_call(
        _node_body,
        grid=(n // nblk,),
        in_specs=[pl.BlockSpec((nblk, d), lambda i: (i, 0)),
                  pl.BlockSpec((_NC, nblk, hd), lambda i: (0, i, 0)),
                  pl.BlockSpec((d, hd), lambda i: (0, 0)),
                  pl.BlockSpec((hd, hd), lambda i: (0, 0)),
                  pl.BlockSpec((1, hd), lambda i: (0, 0)),
                  pl.BlockSpec((hd, d), lambda i: (0, 0)),
                  pl.BlockSpec((1, d), lambda i: (0, 0))],
        out_specs=pl.BlockSpec((nblk, d), lambda i: (i, 0)),
        out_shape=jax.ShapeDtypeStruct((n, d), jnp.float32),
    )(h, agg2, w3at, w3bt, b3r, w4t, b4r)
    return out


# speedup vs baseline: 3.5096x; 3.5096x over previous
import functools

import jax
import jax.numpy as jnp
from jax import lax
from jax.experimental import pallas as pl
from jax.experimental.pallas import tpu as pltpu
from jax.experimental.pallas import tpu_sc as plsc

_NC = 2
_NS = 16
_NW = _NC * _NS


def _silu(x):
    return x * jax.nn.sigmoid(x)


def _proj_body(h_ref, w1at_ref, w1bt_ref, s_ref, t_ref):
    hb = h_ref[...]
    s_ref[...] = jnp.dot(hb, w1at_ref[...], preferred_element_type=jnp.float32)
    t_ref[...] = jnp.dot(hb, w1bt_ref[...], preferred_element_type=jnp.float32)


def _sc_gather(hp_s, hp_t, row, col, *, chunk):
    e = row.shape[0]
    d = hp_s.shape[1]
    epw = e // _NW
    mesh = plsc.VectorSubcoreMesh(core_axis_name="c", subcore_axis_name="s")

    @functools.partial(
        pl.kernel,
        out_type=[jax.ShapeDtypeStruct((e, d), jnp.float32),
                  jax.ShapeDtypeStruct((e, d), jnp.float32)],
        mesh=mesh,
        scratch_types=[pltpu.VMEM((chunk,), jnp.int32),
                       pltpu.VMEM((chunk, d), jnp.float32),
                       pltpu.SemaphoreType.DMA],
    )
    def k(hp_s_hbm, hp_t_hbm, row_hbm, col_hbm, s_hbm, t_hbm, idx_v, rows_v, sem):
        wid = lax.axis_index("s") * _NC + lax.axis_index("c")
        base = wid * epw

        @pl.loop(0, epw, step=chunk)
        def _(k0):
            off = base + k0
            pltpu.sync_copy(row_hbm.at[pl.ds(off, chunk)], idx_v)
            pltpu.async_copy(hp_s_hbm.at[idx_v], rows_v, sem).wait()
            pltpu.sync_copy(rows_v, s_hbm.at[pl.ds(off, chunk)])
            pltpu.sync_copy(col_hbm.at[pl.ds(off, chunk)], idx_v)
            pltpu.async_copy(hp_t_hbm.at[idx_v], rows_v, sem).wait()
            pltpu.sync_copy(rows_v, t_hbm.at[pl.ds(off, chunk)])

    return k(hp_s, hp_t, row, col)


def _edge_body(s_ref, t_ref, cd_ref, w1c_ref, b1_ref, w2t_ref, b2_ref, m_ref):
    cd = cd_ref[...]
    radial = jnp.sum(cd * cd, axis=1, keepdims=True)
    x = s_ref[...] + t_ref[...] + radial * w1c_ref[...] + b1_ref[...]
    x = _silu(x)
    y = jnp.dot(x, w2t_ref[...], preferred_element_type=jnp.float32) + b2_ref[...]
    m_ref[...] = _silu(y)


def _sc_scatter(m, row, zeros, n, *, chunk):
    e, d = m.shape
    epw = e // _NW
    rps = -(-n // _NS)
    rps += (-rps) % 8
    tail = n - (_NS - 1) * rps
    assert tail > 0 and rps % 8 == 0
    mesh = plsc.VectorSubcoreMesh(core_axis_name="c", subcore_axis_name="s")

    @functools.partial(
        pl.kernel,
        out_type=jax.ShapeDtypeStruct((_NC, n, d), jnp.float32),
        mesh=mesh,
        scratch_types=[pltpu.VMEM((chunk,), jnp.int32),
                       pltpu.VMEM((chunk, d), jnp.float32),
                       pltpu.VMEM_SHARED((n, d), jnp.float32),
                       pltpu.SemaphoreType.DMA],
    )
    def k(m_hbm, row_hbm, z_hbm, out_hbm, idx_v, m_v, agg_sh, sem):
        cid = lax.axis_index("c")
        sid = lax.axis_index("s")
        wid = sid * _NC + cid
        base = wid * epw

        @pl.when(sid < _NS - 1)
        def _():
            pltpu.sync_copy(z_hbm.at[pl.ds(sid * rps, rps)],
                            agg_sh.at[pl.ds(sid * rps, rps)])

        @pl.when(sid == _NS - 1)
        def _():
            pltpu.sync_copy(z_hbm.at[pl.ds(sid * rps, tail)],
                            agg_sh.at[pl.ds(sid * rps, tail)])

        plsc.subcore_barrier()

        @pl.loop(0, epw, step=chunk)
        def _(k0):
            off = base + k0
            pltpu.sync_copy(row_hbm.at[pl.ds(off, chunk)], idx_v)
            pltpu.sync_copy(m_hbm.at[pl.ds(off, chunk)], m_v)
            pltpu.sync_copy(m_v, agg_sh.at[idx_v], add=True)

        plsc.subcore_barrier()

        @pl.when(sid < _NS - 1)
        def _():
            pltpu.sync_copy(agg_sh.at[pl.ds(sid * rps, rps)],
                            out_hbm.at[cid].at[pl.ds(sid * rps, rps)])

        @pl.when(sid == _NS - 1)
        def _():
            pltpu.sync_copy(agg_sh.at[pl.ds(sid * rps, tail)],
                            out_hbm.at[cid].at[pl.ds(sid * rps, tail)])

    return k(m, row, zeros)


def _node_body(h_ref, agg_ref, w3at_ref, w3bt_ref, b3_ref,
               w4t_ref, b4_ref, o_ref):
    agg = agg_ref[0] + agg_ref[1]
    x = (jnp.dot(h_ref[...], w3at_ref[...], preferred_element_type=jnp.float32)
         + jnp.dot(agg, w3bt_ref[...], preferred_element_type=jnp.float32)
         + b3_ref[...])
    x = _silu(x)
    o_ref[...] = jnp.dot(x, w4t_ref[...], preferred_element_type=jnp.float32) + b4_ref[...]


def kernel(h, edges_index, coord_diff, w1, b1, w2, b2, w3, b3, w4, b4):
    n, d = h.shape
    e = edges_index.shape[1]
    hd = w1.shape[0]

    ei = edges_index.astype(jnp.int32)
    row = ei[0]
    col = ei[1]

    w1at = w1[:, :d].T
    w1bt = w1[:, d:2 * d].T
    w1c = w1[:, 2 * d].reshape(1, hd)
    b1r = b1.reshape(1, hd)
    w2t = w2.T
    b2r = b2.reshape(1, hd)
    w3at = w3[:, :d].T
    w3bt = w3[:, d:].T
    b3r = b3.reshape(1, hd)
    w4t = w4.T
    b4r = b4.reshape(1, d)

    nblk = 2000
    hp_s, hp_t = pl.pallas_call(
        _proj_body,
        grid=(n // nblk,),
        in_specs=[pl.BlockSpec((nblk, d), lambda i: (i, 0)),
                  pl.BlockSpec((d, hd), lambda i: (0, 0)),
                  pl.BlockSpec((d, hd), lambda i: (0, 0))],
        out_specs=[pl.BlockSpec((nblk, hd), lambda i: (i, 0)),
                   pl.BlockSpec((nblk, hd), lambda i: (i, 0))],
        out_shape=[jax.ShapeDtypeStruct((n, hd), jnp.float32),
                   jax.ShapeDtypeStruct((n, hd), jnp.float32)],
    )(h, w1at, w1bt)

    s_g, t_g = _sc_gather(hp_s, hp_t, row, col, chunk=400)

    eblk = 2000
    m = pl.pallas_call(
        _edge_body,
        grid=(e // eblk,),
        in_specs=[pl.BlockSpec((eblk, hd), lambda i: (i, 0)),
                  pl.BlockSpec((eblk, hd), lambda i: (i, 0)),
                  pl.BlockSpec((eblk, 3), lambda i: (i, 0)),
                  pl.BlockSpec((1, hd), lambda i: (0, 0)),
                  pl.BlockSpec((1, hd), lambda i: (0, 0)),
                  pl.BlockSpec((hd, hd), lambda i: (0, 0)),
                  pl.BlockSpec((1, hd), lambda i: (0, 0))],
        out_specs=pl.BlockSpec((eblk, hd), lambda i: (i, 0)),
        out_shape=jax.ShapeDtypeStruct((e, hd), jnp.float32),
    )(s_g, t_g, coord_diff, w1c, b1r, w2t, b2r)

    zeros = jnp.zeros((n, hd), jnp.float32)
    agg = _sc_scatter(m, row, zeros, n, chunk=80)

    out = pl.pallas_call(
        _node_body,
        grid=(n // nblk,),
        in_specs=[pl.BlockSpec((nblk, d), lambda i: (i, 0)),
                  pl.BlockSpec((_NC, nblk, hd), lambda i: (0, i, 0)),
                  pl.BlockSpec((d, hd), lambda i: (0, 0)),
                  pl.BlockSpec((hd, hd), lambda i: (0, 0)),
                  pl.BlockSpec((1, hd), lambda i: (0, 0)),
                  pl.BlockSpec((hd, d), lambda i: (0, 0)),
                  pl.BlockSpec((1, d), lambda i: (0, 0))],
        out_specs=pl.BlockSpec((nblk, d), lambda i: (i, 0)),
        out_shape=jax.ShapeDtypeStruct((n, d), jnp.float32),
    )(h, agg, w3at, w3bt, b3r, w4t, b4r)
    return out


# speedup vs baseline: 3.6341x; 1.0355x over previous
import functools

import jax
import jax.numpy as jnp
from jax import lax
from jax.experimental import pallas as pl
from jax.experimental.pallas import tpu as pltpu
from jax.experimental.pallas import tpu_sc as plsc

_NC = 2
_NS = 16
_NW = _NC * _NS


def _silu(x):
    return x * jax.nn.sigmoid(x)


def _proj_body(h_ref, w1at_ref, w1bt_ref, s_ref, t_ref):
    hb = h_ref[...]
    s_ref[...] = jnp.dot(hb, w1at_ref[...], preferred_element_type=jnp.float32)
    t_ref[...] = jnp.dot(hb, w1bt_ref[...], preferred_element_type=jnp.float32)


def _sc_gather(hp_s, hp_t, row, col, *, chunk):
    e = row.shape[0]
    d = hp_s.shape[1]
    epw = e // _NW
    nch = epw // chunk
    mesh = plsc.VectorSubcoreMesh(core_axis_name="c", subcore_axis_name="s")

    @functools.partial(
        pl.kernel,
        out_type=[jax.ShapeDtypeStruct((e, d), jnp.float32),
                  jax.ShapeDtypeStruct((e, d), jnp.float32)],
        mesh=mesh,
        scratch_types=[pltpu.VMEM((chunk,), jnp.int32),
                       pltpu.VMEM((chunk,), jnp.int32),
                       pltpu.VMEM((chunk,), jnp.int32),
                       pltpu.VMEM((chunk,), jnp.int32),
                       pltpu.VMEM((chunk, d), jnp.float32),
                       pltpu.VMEM((chunk, d), jnp.float32),
                       pltpu.SemaphoreType.DMA,
                       pltpu.SemaphoreType.DMA,
                       pltpu.SemaphoreType.DMA,
                       pltpu.SemaphoreType.DMA,
                       pltpu.SemaphoreType.DMA,
                       pltpu.SemaphoreType.DMA],
    )
    def k(hp_s_hbm, hp_t_hbm, row_hbm, col_hbm, s_hbm, t_hbm,
          ir0, ic0, ir1, ic1, bs, bt, sir, sic, sgs, sgt, sws, swt):
        wid = lax.axis_index("s") * _NC + lax.axis_index("c")
        base = wid * epw

        def prefetch_idx(ko, ira, ica):
            off = base + ko * chunk
            pltpu.async_copy(row_hbm.at[pl.ds(off, chunk)], ira, sir)
            pltpu.async_copy(col_hbm.at[pl.ds(off, chunk)], ica, sic)

        prefetch_idx(0, ir0, ic0)

        @pl.loop(0, nch, step=2)
        def _(k0):
            prefetch_idx(k0 + 1, ir1, ic1)

            @pl.when(k0 > 0)
            def _():
                pltpu.make_async_copy(bs, s_hbm.at[pl.ds(base + (k0 - 1) * chunk, chunk)],
                                      sws).wait()
                pltpu.make_async_copy(bt, t_hbm.at[pl.ds(base + (k0 - 1) * chunk, chunk)],
                                      swt).wait()
            _emit_gather_pair(k0, ir0, ic0, bs, bt,
                              hp_s_hbm, hp_t_hbm, row_hbm, col_hbm,
                              s_hbm, t_hbm, sir, sic, sgs, sgt, sws, swt,
                              base, chunk)

            @pl.when(k0 + 2 < nch)
            def _():
                prefetch_idx(k0 + 2, ir0, ic0)

            pltpu.make_async_copy(bs, s_hbm.at[pl.ds(base + k0 * chunk, chunk)],
                                  sws).wait()
            pltpu.make_async_copy(bt, t_hbm.at[pl.ds(base + k0 * chunk, chunk)],
                                  swt).wait()
            _emit_gather_pair(k0 + 1, ir1, ic1, bs, bt,
                              hp_s_hbm, hp_t_hbm, row_hbm, col_hbm,
                              s_hbm, t_hbm, sir, sic, sgs, sgt, sws, swt,
                              base, chunk)

        last = base + (nch - 1) * chunk
        pltpu.make_async_copy(bs, s_hbm.at[pl.ds(last, chunk)], sws).wait()
        pltpu.make_async_copy(bt, t_hbm.at[pl.ds(last, chunk)], swt).wait()

    return k(hp_s, hp_t, row, col)


def _emit_gather_pair(ko, ira, ica, bs, bt, hp_s_hbm, hp_t_hbm,
                      row_hbm, col_hbm, s_hbm, t_hbm,
                      sir, sic, sgs, sgt, sws, swt, base, chunk):
    off = base + ko * chunk
    pltpu.make_async_copy(row_hbm.at[pl.ds(off, chunk)], ira, sir).wait()
    pltpu.make_async_copy(col_hbm.at[pl.ds(off, chunk)], ica, sic).wait()
    pltpu.async_copy(hp_s_hbm.at[ira], bs, sgs)
    pltpu.async_copy(hp_t_hbm.at[ica], bt, sgt)
    pltpu.make_async_copy(hp_s_hbm.at[ira], bs, sgs).wait()
    pltpu.async_copy(bs, s_hbm.at[pl.ds(off, chunk)], sws)
    pltpu.make_async_copy(hp_t_hbm.at[ica], bt, sgt).wait()
    pltpu.async_copy(bt, t_hbm.at[pl.ds(off, chunk)], swt)


def _edge_body(s_ref, t_ref, cd_ref, w1c_ref, b1_ref, w2t_ref, b2_ref, m_ref):
    cd = cd_ref[...]
    radial = jnp.sum(cd * cd, axis=1, keepdims=True)
    x = s_ref[...] + t_ref[...] + radial * w1c_ref[...] + b1_ref[...]
    x = _silu(x)
    y = jnp.dot(x, w2t_ref[...], preferred_element_type=jnp.float32) + b2_ref[...]
    m_ref[...] = _silu(y)


def _sc_scatter(m, row, zeros, n, *, chunk):
    e, d = m.shape
    epw = e // _NW
    rps = -(-n // _NS)
    rps += (-rps) % 8
    tail = n - (_NS - 1) * rps
    assert tail > 0 and rps % 8 == 0
    mesh = plsc.VectorSubcoreMesh(core_axis_name="c", subcore_axis_name="s")

    @functools.partial(
        pl.kernel,
        out_type=jax.ShapeDtypeStruct((_NC, n, d), jnp.float32),
        mesh=mesh,
        scratch_types=[pltpu.VMEM((chunk,), jnp.int32),
                       pltpu.VMEM((chunk, d), jnp.float32),
                       pltpu.VMEM_SHARED((n, d), jnp.float32),
                       pltpu.SemaphoreType.DMA],
    )
    def k(m_hbm, row_hbm, z_hbm, out_hbm, idx_v, m_v, agg_sh, sem):
        cid = lax.axis_index("c")
        sid = lax.axis_index("s")
        wid = sid * _NC + cid
        base = wid * epw

        @pl.when(sid < _NS - 1)
        def _():
            pltpu.sync_copy(z_hbm.at[pl.ds(sid * rps, rps)],
                            agg_sh.at[pl.ds(sid * rps, rps)])

        @pl.when(sid == _NS - 1)
        def _():
            pltpu.sync_copy(z_hbm.at[pl.ds(sid * rps, tail)],
                            agg_sh.at[pl.ds(sid * rps, tail)])

        plsc.subcore_barrier()

        @pl.loop(0, epw, step=chunk)
        def _(k0):
            off = base + k0
            pltpu.sync_copy(row_hbm.at[pl.ds(off, chunk)], idx_v)
            pltpu.sync_copy(m_hbm.at[pl.ds(off, chunk)], m_v)
            pltpu.sync_copy(m_v, agg_sh.at[idx_v], add=True)

        plsc.subcore_barrier()

        @pl.when(sid < _NS - 1)
        def _():
            pltpu.sync_copy(agg_sh.at[pl.ds(sid * rps, rps)],
                            out_hbm.at[cid].at[pl.ds(sid * rps, rps)])

        @pl.when(sid == _NS - 1)
        def _():
            pltpu.sync_copy(agg_sh.at[pl.ds(sid * rps, tail)],
                            out_hbm.at[cid].at[pl.ds(sid * rps, tail)])

    return k(m, row, zeros)


def _node_body(h_ref, agg_ref, w3at_ref, w3bt_ref, b3_ref,
               w4t_ref, b4_ref, o_ref):
    agg = agg_ref[0] + agg_ref[1]
    x = (jnp.dot(h_ref[...], w3at_ref[...], preferred_element_type=jnp.float32)
         + jnp.dot(agg, w3bt_ref[...], preferred_element_type=jnp.float32)
         + b3_ref[...])
    x = _silu(x)
    o_ref[...] = jnp.dot(x, w4t_ref[...], preferred_element_type=jnp.float32) + b4_ref[...]


def kernel(h, edges_index, coord_diff, w1, b1, w2, b2, w3, b3, w4, b4):
    n, d = h.shape
    e = edges_index.shape[1]
    hd = w1.shape[0]

    ei = edges_index.astype(jnp.int32)
    row = ei[0]
    col = ei[1]

    w1at = w1[:, :d].T
    w1bt = w1[:, d:2 * d].T
    w1c = w1[:, 2 * d].reshape(1, hd)
    b1r = b1.reshape(1, hd)
    w2t = w2.T
    b2r = b2.reshape(1, hd)
    w3at = w3[:, :d].T
    w3bt = w3[:, d:].T
    b3r = b3.reshape(1, hd)
    w4t = w4.T
    b4r = b4.reshape(1, d)

    nblk = 2000
    hp_s, hp_t = pl.pallas_call(
        _proj_body,
        grid=(n // nblk,),
        in_specs=[pl.BlockSpec((nblk, d), lambda i: (i, 0)),
                  pl.BlockSpec((d, hd), lambda i: (0, 0)),
                  pl.BlockSpec((d, hd), lambda i: (0, 0))],
        out_specs=[pl.BlockSpec((nblk, hd), lambda i: (i, 0)),
                   pl.BlockSpec((nblk, hd), lambda i: (i, 0))],
        out_shape=[jax.ShapeDtypeStruct((n, hd), jnp.float32),
                   jax.ShapeDtypeStruct((n, hd), jnp.float32)],
    )(h, w1at, w1bt)

    s_g, t_g = _sc_gather(hp_s, hp_t, row, col, chunk=200)

    eblk = 2000
    m = pl.pallas_call(
        _edge_body,
        grid=(e // eblk,),
        in_specs=[pl.BlockSpec((eblk, hd), lambda i: (i, 0)),
                  pl.BlockSpec((eblk, hd), lambda i: (i, 0)),
                  pl.BlockSpec((eblk, 3), lambda i: (i, 0)),
                  pl.BlockSpec((1, hd), lambda i: (0, 0)),
                  pl.BlockSpec((1, hd), lambda i: (0, 0)),
                  pl.BlockSpec((hd, hd), lambda i: (0, 0)),
                  pl.BlockSpec((1, hd), lambda i: (0, 0))],
        out_specs=pl.BlockSpec((eblk, hd), lambda i: (i, 0)),
        out_shape=jax.ShapeDtypeStruct((e, hd), jnp.float32),
    )(s_g, t_g, coord_diff, w1c, b1r, w2t, b2r)

    zeros = jnp.zeros((n, hd), jnp.float32)
    agg = _sc_scatter(m, row, zeros, n, chunk=80)

    out = pl.pallas_call(
        _node_body,
        grid=(n // nblk,),
        in_specs=[pl.BlockSpec((nblk, d), lambda i: (i, 0)),
                  pl.BlockSpec((_NC, nblk, hd), lambda i: (0, i, 0)),
                  pl.BlockSpec((d, hd), lambda i: (0, 0)),
                  pl.BlockSpec((hd, hd), lambda i: (0, 0)),
                  pl.BlockSpec((1, hd), lambda i: (0, 0)),
                  pl.BlockSpec((hd, d), lambda i: (0, 0)),
                  pl.BlockSpec((1, d), lambda i: (0, 0))],
        out_specs=pl.BlockSpec((nblk, d), lambda i: (i, 0)),
        out_shape=jax.ShapeDtypeStruct((n, d), jnp.float32),
    )(h, agg, w3at, w3bt, b3r, w4t, b4r)
    return out


# speedup vs baseline: 4.6039x; 1.2669x over previous
import functools

import jax
import jax.numpy as jnp
from jax import lax
from jax.experimental import pallas as pl
from jax.experimental.pallas import tpu as pltpu
from jax.experimental.pallas import tpu_sc as plsc

_NC = 2
_NS = 16
_NW = _NC * _NS


def _silu(x):
    return x * jax.nn.sigmoid(x)


def _proj_body(h_ref, w1at_ref, w1bt_ref, s_ref, t_ref):
    hb = h_ref[...]
    s_ref[...] = jnp.dot(hb, w1at_ref[...], preferred_element_type=jnp.float32)
    t_ref[...] = jnp.dot(hb, w1bt_ref[...], preferred_element_type=jnp.float32)


def _sc_gather(hp_s, hp_t, row, col, *, chunk):
    e = row.shape[0]
    n, d = hp_s.shape
    epc = e // _NS
    nch = epc // chunk
    assert nch % 2 == 0 and nch * chunk == epc
    rps = -(-n // _NS)
    rps += (-rps) % 8
    tail = n - (_NS - 1) * rps
    assert tail > 0 and rps % 8 == 0
    mesh = plsc.VectorSubcoreMesh(core_axis_name="c", subcore_axis_name="s")

    @functools.partial(
        pl.kernel,
        out_type=[jax.ShapeDtypeStruct((e, d), jnp.float32),
                  jax.ShapeDtypeStruct((e, d), jnp.float32)],
        mesh=mesh,
        scratch_types=[pltpu.VMEM((epc,), jnp.int32),
                       pltpu.VMEM((chunk, d), jnp.float32),
                       pltpu.VMEM((chunk, d), jnp.float32),
                       pltpu.VMEM_SHARED((n, d), jnp.float32),
                       pltpu.SemaphoreType.DMA,
                       pltpu.SemaphoreType.DMA,
                       pltpu.SemaphoreType.DMA,
                       pltpu.SemaphoreType.DMA],
    )
    def k(hp_s_hbm, hp_t_hbm, row_hbm, col_hbm, s_hbm, t_hbm,
          idx_v, b0, b1, hp_sh, sg0, sg1, sw0, sw1):
        cid = lax.axis_index("c")
        sid = lax.axis_index("s")
        base = sid * epc

        def load_hp(src_hbm):
            @pl.when(sid < _NS - 1)
            def _():
                pltpu.sync_copy(src_hbm.at[pl.ds(sid * rps, rps)],
                                hp_sh.at[pl.ds(sid * rps, rps)])

            @pl.when(sid == _NS - 1)
            def _():
                pltpu.sync_copy(src_hbm.at[pl.ds(sid * rps, tail)],
                                hp_sh.at[pl.ds(sid * rps, tail)])

        def serve(idx_hbm, out_hbm):
            pltpu.sync_copy(idx_hbm.at[pl.ds(base, epc)], idx_v)

            def gather(ko, buf, sg):
                pltpu.async_copy(hp_sh.at[idx_v.at[pl.ds(ko * chunk, chunk)]],
                                 buf, sg)

            def gather_wait(ko, buf, sg):
                pltpu.make_async_copy(
                    hp_sh.at[idx_v.at[pl.ds(ko * chunk, chunk)]], buf, sg
                ).wait()

            def wb(ko, buf, sw):
                pltpu.async_copy(buf, out_hbm.at[pl.ds(base + ko * chunk, chunk)],
                                 sw)

            def wb_wait(ko, buf, sw):
                pltpu.make_async_copy(
                    buf, out_hbm.at[pl.ds(base + ko * chunk, chunk)], sw
                ).wait()

            gather(0, b0, sg0)

            @pl.loop(0, nch, step=2)
            def _(k0):
                gather(k0 + 1, b1, sg1)
                gather_wait(k0, b0, sg0)
                wb(k0, b0, sw0)
                gather_wait(k0 + 1, b1, sg1)
                wb(k0 + 1, b1, sw1)
                wb_wait(k0, b0, sw0)

                @pl.when(k0 + 2 < nch)
                def _():
                    gather(k0 + 2, b0, sg0)

                wb_wait(k0 + 1, b1, sw1)

        @pl.when(cid == 0)
        def _():
            load_hp(hp_s_hbm)

        @pl.when(cid == 1)
        def _():
            load_hp(hp_t_hbm)

        plsc.subcore_barrier()

        @pl.when(cid == 0)
        def _():
            serve(row_hbm, s_hbm)

        @pl.when(cid == 1)
        def _():
            serve(col_hbm, t_hbm)

    return k(hp_s, hp_t, row, col)


def _edge_body(s_ref, t_ref, cd_ref, w1c_ref, b1_ref, w2t_ref, b2_ref, m_ref):
    cd = cd_ref[...]
    radial = jnp.sum(cd * cd, axis=1, keepdims=True)
    x = s_ref[...] + t_ref[...] + radial * w1c_ref[...] + b1_ref[...]
    x = _silu(x)
    y = jnp.dot(x, w2t_ref[...], preferred_element_type=jnp.float32) + b2_ref[...]
    m_ref[...] = _silu(y)


def _sc_scatter(m, row, zeros, n, *, chunk):
    e, d = m.shape
    epw = e // _NW
    rps = -(-n // _NS)
    rps += (-rps) % 8
    tail = n - (_NS - 1) * rps
    assert tail > 0 and rps % 8 == 0
    mesh = plsc.VectorSubcoreMesh(core_axis_name="c", subcore_axis_name="s")

    @functools.partial(
        pl.kernel,
        out_type=jax.ShapeDtypeStruct((_NC, n, d), jnp.float32),
        mesh=mesh,
        scratch_types=[pltpu.VMEM((chunk,), jnp.int32),
                       pltpu.VMEM((chunk,), jnp.int32),
                       pltpu.VMEM((chunk, d), jnp.float32),
                       pltpu.VMEM((chunk, d), jnp.float32),
                       pltpu.VMEM_SHARED((n, d), jnp.float32),
                       pltpu.SemaphoreType.DMA,
                       pltpu.SemaphoreType.DMA,
                       pltpu.SemaphoreType.DMA,
                       pltpu.SemaphoreType.DMA],
    )
    def k(m_hbm, row_hbm, z_hbm, out_hbm, i0, i1, m0, m1, agg_sh,
          si0, si1, sm0, sm1):
        cid = lax.axis_index("c")
        sid = lax.axis_index("s")
        wid = sid * _NC + cid
        base = wid * epw
        nch = epw // chunk

        @pl.when(sid < _NS - 1)
        def _():
            pltpu.sync_copy(z_hbm.at[pl.ds(sid * rps, rps)],
                            agg_sh.at[pl.ds(sid * rps, rps)])

        @pl.when(sid == _NS - 1)
        def _():
            pltpu.sync_copy(z_hbm.at[pl.ds(sid * rps, tail)],
                            agg_sh.at[pl.ds(sid * rps, tail)])

        plsc.subcore_barrier()

        def prefetch(ko, iv, mv, si, sm):
            off = base + ko * chunk
            pltpu.async_copy(row_hbm.at[pl.ds(off, chunk)], iv, si)
            pltpu.async_copy(m_hbm.at[pl.ds(off, chunk)], mv, sm)

        def scatter(ko, iv, mv, si, sm):
            off = base + ko * chunk
            pltpu.make_async_copy(row_hbm.at[pl.ds(off, chunk)], iv, si).wait()
            pltpu.make_async_copy(m_hbm.at[pl.ds(off, chunk)], mv, sm).wait()
            pltpu.sync_copy(mv, agg_sh.at[iv], add=True)

        prefetch(0, i0, m0, si0, sm0)

        @pl.loop(0, nch, step=2)
        def _(k0):
            prefetch(k0 + 1, i1, m1, si1, sm1)
            scatter(k0, i0, m0, si0, sm0)

            @pl.when(k0 + 2 < nch)
            def _():
                prefetch(k0 + 2, i0, m0, si0, sm0)

            scatter(k0 + 1, i1, m1, si1, sm1)

        plsc.subcore_barrier()

        @pl.when(sid < _NS - 1)
        def _():
            pltpu.sync_copy(agg_sh.at[pl.ds(sid * rps, rps)],
                            out_hbm.at[cid].at[pl.ds(sid * rps, rps)])

        @pl.when(sid == _NS - 1)
        def _():
            pltpu.sync_copy(agg_sh.at[pl.ds(sid * rps, tail)],
                            out_hbm.at[cid].at[pl.ds(sid * rps, tail)])

    return k(m, row, zeros)


def _node_body(h_ref, agg_ref, w3at_ref, w3bt_ref, b3_ref,
               w4t_ref, b4_ref, o_ref):
    agg = agg_ref[0] + agg_ref[1]
    x = (jnp.dot(h_ref[...], w3at_ref[...], preferred_element_type=jnp.float32)
         + jnp.dot(agg, w3bt_ref[...], preferred_element_type=jnp.float32)
         + b3_ref[...])
    x = _silu(x)
    o_ref[...] = jnp.dot(x, w4t_ref[...], preferred_element_type=jnp.float32) + b4_ref[...]


def kernel(h, edges_index, coord_diff, w1, b1, w2, b2, w3, b3, w4, b4):
    n, d = h.shape
    e = edges_index.shape[1]
    hd = w1.shape[0]

    ei = edges_index.astype(jnp.int32)
    row = ei[0]
    col = ei[1]

    w1at = w1[:, :d].T
    w1bt = w1[:, d:2 * d].T
    w1c = w1[:, 2 * d].reshape(1, hd)
    b1r = b1.reshape(1, hd)
    w2t = w2.T
    b2r = b2.reshape(1, hd)
    w3at = w3[:, :d].T
    w3bt = w3[:, d:].T
    b3r = b3.reshape(1, hd)
    w4t = w4.T
    b4r = b4.reshape(1, d)

    nblk = 2000
    hp_s, hp_t = pl.pallas_call(
        _proj_body,
        grid=(n // nblk,),
        in_specs=[pl.BlockSpec((nblk, d), lambda i: (i, 0)),
                  pl.BlockSpec((d, hd), lambda i: (0, 0)),
                  pl.BlockSpec((d, hd), lambda i: (0, 0))],
        out_specs=[pl.BlockSpec((nblk, hd), lambda i: (i, 0)),
                   pl.BlockSpec((nblk, hd), lambda i: (i, 0))],
        out_shape=[jax.ShapeDtypeStruct((n, hd), jnp.float32),
                   jax.ShapeDtypeStruct((n, hd), jnp.float32)],
    )(h, w1at, w1bt)

    s_g, t_g = _sc_gather(hp_s, hp_t, row, col, chunk=80)

    eblk = 2000
    m = pl.pallas_call(
        _edge_body,
        grid=(e // eblk,),
        in_specs=[pl.BlockSpec((eblk, hd), lambda i: (i, 0)),
                  pl.BlockSpec((eblk, hd), lambda i: (i, 0)),
                  pl.BlockSpec((eblk, 3), lambda i: (i, 0)),
                  pl.BlockSpec((1, hd), lambda i: (0, 0)),
                  pl.BlockSpec((1, hd), lambda i: (0, 0)),
                  pl.BlockSpec((hd, hd), lambda i: (0, 0)),
                  pl.BlockSpec((1, hd), lambda i: (0, 0))],
        out_specs=pl.BlockSpec((eblk, hd), lambda i: (i, 0)),
        out_shape=jax.ShapeDtypeStruct((e, hd), jnp.float32),
    )(s_g, t_g, coord_diff, w1c, b1r, w2t, b2r)

    zeros = jnp.zeros((n, hd), jnp.float32)
    agg = _sc_scatter(m, row, zeros, n, chunk=40)

    out = pl.pallas_call(
        _node_body,
        grid=(n // nblk,),
        in_specs=[pl.BlockSpec((nblk, d), lambda i: (i, 0)),
                  pl.BlockSpec((_NC, nblk, hd), lambda i: (0, i, 0)),
                  pl.BlockSpec((d, hd), lambda i: (0, 0)),
                  pl.BlockSpec((hd, hd), lambda i: (0, 0)),
                  pl.BlockSpec((1, hd), lambda i: (0, 0)),
                  pl.BlockSpec((hd, d), lambda i: (0, 0)),
                  pl.BlockSpec((1, d), lambda i: (0, 0))],
        out_specs=pl.BlockSpec((nblk, d), lambda i: (i, 0)),
        out_shape=jax.ShapeDtypeStruct((n, d), jnp.float32),
    )(h, agg, w3at, w3bt, b3r, w4t, b4r)
    return out


# speedup vs baseline: 5.1250x; 1.1132x over previous
import functools

import jax
import jax.numpy as jnp
from jax import lax
from jax.experimental import pallas as pl
from jax.experimental.pallas import tpu as pltpu
from jax.experimental.pallas import tpu_sc as plsc

_NC = 2
_NS = 16
_NW = _NC * _NS


def _silu(x):
    return x * jax.nn.sigmoid(x)


def _proj_body(h_ref, w1at_ref, w1bt_ref, s_ref, t_ref):
    hb = h_ref[...]
    s_ref[...] = jnp.dot(hb, w1at_ref[...], preferred_element_type=jnp.float32)
    t_ref[...] = jnp.dot(hb, w1bt_ref[...], preferred_element_type=jnp.float32)


def _sc_gather(hp_s, hp_t, row, col, *, chunk):
    e = row.shape[0]
    n, d = hp_s.shape
    epc = e // _NS
    nch = epc // chunk
    assert nch * chunk == epc
    nch_main = nch - (nch % 2)
    rps = -(-n // _NS)
    rps += (-rps) % 8
    tail = n - (_NS - 1) * rps
    assert tail > 0 and rps % 8 == 0
    mesh = plsc.VectorSubcoreMesh(core_axis_name="c", subcore_axis_name="s")

    @functools.partial(
        pl.kernel,
        out_type=[jax.ShapeDtypeStruct((e, d), jnp.float32),
                  jax.ShapeDtypeStruct((e, d), jnp.float32)],
        mesh=mesh,
        scratch_types=[pltpu.VMEM((epc,), jnp.int32),
                       pltpu.VMEM((chunk, d), jnp.float32),
                       pltpu.VMEM((chunk, d), jnp.float32),
                       pltpu.VMEM_SHARED((n, d), jnp.float32),
                       pltpu.SemaphoreType.DMA,
                       pltpu.SemaphoreType.DMA,
                       pltpu.SemaphoreType.DMA,
                       pltpu.SemaphoreType.DMA],
    )
    def k(hp_s_hbm, hp_t_hbm, row_hbm, col_hbm, s_hbm, t_hbm,
          idx_v, b0, b1, hp_sh, sg0, sg1, sw0, sw1):
        cid = lax.axis_index("c")
        sid = lax.axis_index("s")
        base = sid * epc

        def load_hp(src_hbm):
            @pl.when(sid < _NS - 1)
            def _():
                pltpu.sync_copy(src_hbm.at[pl.ds(sid * rps, rps)],
                                hp_sh.at[pl.ds(sid * rps, rps)])

            @pl.when(sid == _NS - 1)
            def _():
                pltpu.sync_copy(src_hbm.at[pl.ds(sid * rps, tail)],
                                hp_sh.at[pl.ds(sid * rps, tail)])

        def serve(idx_hbm, out_hbm):
            pltpu.sync_copy(idx_hbm.at[pl.ds(base, epc)], idx_v)

            def gather(ko, buf, sg):
                pltpu.async_copy(hp_sh.at[idx_v.at[pl.ds(ko * chunk, chunk)]],
                                 buf, sg)

            def gather_wait(ko, buf, sg):
                pltpu.make_async_copy(
                    hp_sh.at[idx_v.at[pl.ds(ko * chunk, chunk)]], buf, sg
                ).wait()

            def wb(ko, buf, sw):
                pltpu.async_copy(buf, out_hbm.at[pl.ds(base + ko * chunk, chunk)],
                                 sw)

            def wb_wait(ko, buf, sw):
                pltpu.make_async_copy(
                    buf, out_hbm.at[pl.ds(base + ko * chunk, chunk)], sw
                ).wait()

            gather(0, b0, sg0)

            @pl.loop(0, nch_main, step=2)
            def _(k0):
                gather(k0 + 1, b1, sg1)
                gather_wait(k0, b0, sg0)
                wb(k0, b0, sw0)
                gather_wait(k0 + 1, b1, sg1)
                wb(k0 + 1, b1, sw1)
                wb_wait(k0, b0, sw0)

                @pl.when(k0 + 2 < nch)
                def _():
                    gather(k0 + 2, b0, sg0)

                wb_wait(k0 + 1, b1, sw1)

            if nch % 2:
                gather_wait(nch - 1, b0, sg0)
                wb(nch - 1, b0, sw0)
                wb_wait(nch - 1, b0, sw0)

        @pl.when(cid == 0)
        def _():
            load_hp(hp_s_hbm)

        @pl.when(cid == 1)
        def _():
            load_hp(hp_t_hbm)

        plsc.subcore_barrier()

        @pl.when(cid == 0)
        def _():
            serve(row_hbm, s_hbm)

        @pl.when(cid == 1)
        def _():
            serve(col_hbm, t_hbm)

    return k(hp_s, hp_t, row, col)


def _edge_body(s_ref, t_ref, cd_ref, w1c_ref, b1_ref, w2t_ref, b2_ref, m_ref):
    cd = cd_ref[...]
    radial = jnp.sum(cd * cd, axis=1, keepdims=True)
    x = s_ref[...] + t_ref[...] + radial * w1c_ref[...] + b1_ref[...]
    x = _silu(x)
    y = jnp.dot(x, w2t_ref[...], preferred_element_type=jnp.float32) + b2_ref[...]
    m_ref[...] = _silu(y)


def _sc_scatter(m, row, zeros, n, *, chunk):
    e, d = m.shape
    epw = e // _NW
    rps = -(-n // _NS)
    rps += (-rps) % 8
    tail = n - (_NS - 1) * rps
    assert tail > 0 and rps % 8 == 0
    mesh = plsc.VectorSubcoreMesh(core_axis_name="c", subcore_axis_name="s")

    @functools.partial(
        pl.kernel,
        out_type=jax.ShapeDtypeStruct((_NC, n, d), jnp.float32),
        mesh=mesh,
        scratch_types=[pltpu.VMEM((chunk,), jnp.int32),
                       pltpu.VMEM((chunk,), jnp.int32),
                       pltpu.VMEM((chunk, d), jnp.float32),
                       pltpu.VMEM((chunk, d), jnp.float32),
                       pltpu.VMEM_SHARED((n, d), jnp.float32),
                       pltpu.SemaphoreType.DMA,
                       pltpu.SemaphoreType.DMA,
                       pltpu.SemaphoreType.DMA,
                       pltpu.SemaphoreType.DMA],
    )
    def k(m_hbm, row_hbm, z_hbm, out_hbm, i0, i1, m0, m1, agg_sh,
          si0, si1, sm0, sm1):
        cid = lax.axis_index("c")
        sid = lax.axis_index("s")
        wid = sid * _NC + cid
        base = wid * epw
        nch = epw // chunk

        @pl.when(sid < _NS - 1)
        def _():
            pltpu.sync_copy(z_hbm.at[pl.ds(sid * rps, rps)],
                            agg_sh.at[pl.ds(sid * rps, rps)])

        @pl.when(sid == _NS - 1)
        def _():
            pltpu.sync_copy(z_hbm.at[pl.ds(sid * rps, tail)],
                            agg_sh.at[pl.ds(sid * rps, tail)])

        plsc.subcore_barrier()

        def prefetch(ko, iv, mv, si, sm):
            off = base + ko * chunk
            pltpu.async_copy(row_hbm.at[pl.ds(off, chunk)], iv, si)
            pltpu.async_copy(m_hbm.at[pl.ds(off, chunk)], mv, sm)

        def scatter(ko, iv, mv, si, sm):
            off = base + ko * chunk
            pltpu.make_async_copy(row_hbm.at[pl.ds(off, chunk)], iv, si).wait()
            pltpu.make_async_copy(m_hbm.at[pl.ds(off, chunk)], mv, sm).wait()
            pltpu.sync_copy(mv, agg_sh.at[iv], add=True)

        prefetch(0, i0, m0, si0, sm0)
        nch_main = nch - (nch % 2)

        @pl.loop(0, nch_main, step=2)
        def _(k0):
            prefetch(k0 + 1, i1, m1, si1, sm1)
            scatter(k0, i0, m0, si0, sm0)

            @pl.when(k0 + 2 < nch)
            def _():
                prefetch(k0 + 2, i0, m0, si0, sm0)

            scatter(k0 + 1, i1, m1, si1, sm1)

        if nch % 2:
            scatter(nch - 1, i0, m0, si0, sm0)

        plsc.subcore_barrier()

        @pl.when(sid < _NS - 1)
        def _():
            pltpu.sync_copy(agg_sh.at[pl.ds(sid * rps, rps)],
                            out_hbm.at[cid].at[pl.ds(sid * rps, rps)])

        @pl.when(sid == _NS - 1)
        def _():
            pltpu.sync_copy(agg_sh.at[pl.ds(sid * rps, tail)],
                            out_hbm.at[cid].at[pl.ds(sid * rps, tail)])

    return k(m, row, zeros)


def _node_body(h_ref, agg_a_ref, agg_b_ref, w3at_ref, w3bt_ref, b3_ref,
               w4t_ref, b4_ref, o_ref):
    agg = (agg_a_ref[0] + agg_a_ref[1]) + (agg_b_ref[0] + agg_b_ref[1])
    x = (jnp.dot(h_ref[...], w3at_ref[...], preferred_element_type=jnp.float32)
         + jnp.dot(agg, w3bt_ref[...], preferred_element_type=jnp.float32)
         + b3_ref[...])
    x = _silu(x)
    o_ref[...] = jnp.dot(x, w4t_ref[...], preferred_element_type=jnp.float32) + b4_ref[...]


def kernel(h, edges_index, coord_diff, w1, b1, w2, b2, w3, b3, w4, b4):
    n, d = h.shape
    e = edges_index.shape[1]
    hd = w1.shape[0]

    ei = edges_index.astype(jnp.int32)
    row = ei[0]
    col = ei[1]

    w1at = w1[:, :d].T
    w1bt = w1[:, d:2 * d].T
    w1c = w1[:, 2 * d].reshape(1, hd)
    b1r = b1.reshape(1, hd)
    w2t = w2.T
    b2r = b2.reshape(1, hd)
    w3at = w3[:, :d].T
    w3bt = w3[:, d:].T
    b3r = b3.reshape(1, hd)
    w4t = w4.T
    b4r = b4.reshape(1, d)

    nblk = 2000
    hp_s, hp_t = pl.pallas_call(
        _proj_body,
        grid=(n // nblk,),
        in_specs=[pl.BlockSpec((nblk, d), lambda i: (i, 0)),
                  pl.BlockSpec((d, hd), lambda i: (0, 0)),
                  pl.BlockSpec((d, hd), lambda i: (0, 0))],
        out_specs=[pl.BlockSpec((nblk, hd), lambda i: (i, 0)),
                   pl.BlockSpec((nblk, hd), lambda i: (i, 0))],
        out_shape=[jax.ShapeDtypeStruct((n, hd), jnp.float32),
                   jax.ShapeDtypeStruct((n, hd), jnp.float32)],
    )(h, w1at, w1bt)

    zeros = jnp.zeros((n, hd), jnp.float32)
    eblk = 2000

    def half(row_h, col_h, cd_h):
        eh = row_h.shape[0]
        s_g, t_g = _sc_gather(hp_s, hp_t, row_h, col_h, chunk=80)
        m = pl.pallas_call(
            _edge_body,
            grid=(eh // eblk,),
            in_specs=[pl.BlockSpec((eblk, hd), lambda i: (i, 0)),
                      pl.BlockSpec((eblk, hd), lambda i: (i, 0)),
                      pl.BlockSpec((eblk, 3), lambda i: (i, 0)),
                      pl.BlockSpec((1, hd), lambda i: (0, 0)),
                      pl.BlockSpec((1, hd), lambda i: (0, 0)),
                      pl.BlockSpec((hd, hd), lambda i: (0, 0)),
                      pl.BlockSpec((1, hd), lambda i: (0, 0))],
            out_specs=pl.BlockSpec((eblk, hd), lambda i: (i, 0)),
            out_shape=jax.ShapeDtypeStruct((eh, hd), jnp.float32),
        )(s_g, t_g, cd_h, w1c, b1r, w2t, b2r)
        return _sc_scatter(m, row_h, zeros, n, chunk=40)

    e2 = e // 2
    agg_a = half(row[:e2], col[:e2], coord_diff[:e2])
    agg_b = half(row[e2:], col[e2:], coord_diff[e2:])

    out = pl.pallas_call(
        _node_body,
        grid=(n // nblk,),
        in_specs=[pl.BlockSpec((nblk, d), lambda i: (i, 0)),
                  pl.BlockSpec((_NC, nblk, hd), lambda i: (0, i, 0)),
                  pl.BlockSpec((_NC, nblk, hd), lambda i: (0, i, 0)),
                  pl.BlockSpec((d, hd), lambda i: (0, 0)),
                  pl.BlockSpec((hd, hd), lambda i: (0, 0)),
                  pl.BlockSpec((1, hd), lambda i: (0, 0)),
                  pl.BlockSpec((hd, d), lambda i: (0, 0)),
                  pl.BlockSpec((1, d), lambda i: (0, 0))],
        out_specs=pl.BlockSpec((nblk, d), lambda i: (i, 0)),
        out_shape=jax.ShapeDtypeStruct((n, d), jnp.float32),
    )(h, agg_a, agg_b, w3at, w3bt, b3r, w4t, b4r)
    return out


# speedup vs baseline: 5.5074x; 1.0746x over previous
import functools

import jax
import jax.numpy as jnp
from jax import lax
from jax.experimental import pallas as pl
from jax.experimental.pallas import tpu as pltpu
from jax.experimental.pallas import tpu_sc as plsc

_NC = 2
_NS = 16
_NW = _NC * _NS
_SNB = 4


def _silu(x):
    return x * jax.nn.sigmoid(x)


def _proj_body(h_ref, w1at_ref, w1bt_ref, s_ref, t_ref):
    hb = h_ref[...]
    s_ref[...] = jnp.dot(hb, w1at_ref[...], preferred_element_type=jnp.float32)
    t_ref[...] = jnp.dot(hb, w1bt_ref[...], preferred_element_type=jnp.float32)


def _sc_gather(hp_s, hp_t, row, col, *, chunk):
    e = row.shape[0]
    n, d = hp_s.shape
    epc = e // _NS
    nch = epc // chunk
    assert nch * chunk == epc
    rps = -(-n // _NS)
    rps += (-rps) % 8
    tail = n - (_NS - 1) * rps
    assert tail > 0 and rps % 8 == 0
    mesh = plsc.VectorSubcoreMesh(core_axis_name="c", subcore_axis_name="s")

    nb = 3

    @functools.partial(
        pl.kernel,
        out_type=[jax.ShapeDtypeStruct((e, d), jnp.float32),
                  jax.ShapeDtypeStruct((e, d), jnp.float32)],
        mesh=mesh,
        scratch_types=[pltpu.VMEM((epc,), jnp.int32)]
                      + [pltpu.VMEM((chunk, d), jnp.float32)] * nb
                      + [pltpu.VMEM_SHARED((n, d), jnp.float32)]
                      + [pltpu.SemaphoreType.DMA] * (2 * nb),
    )
    def k(hp_s_hbm, hp_t_hbm, row_hbm, col_hbm, s_hbm, t_hbm,
          idx_v, *rest):
        bufs = rest[:nb]
        hp_sh = rest[nb]
        sgs = rest[nb + 1:nb + 1 + nb]
        sws = rest[nb + 1 + nb:]
        cid = lax.axis_index("c")
        sid = lax.axis_index("s")
        base = sid * epc

        def load_hp(src_hbm):
            @pl.when(sid < _NS - 1)
            def _():
                pltpu.sync_copy(src_hbm.at[pl.ds(sid * rps, rps)],
                                hp_sh.at[pl.ds(sid * rps, rps)])

            @pl.when(sid == _NS - 1)
            def _():
                pltpu.sync_copy(src_hbm.at[pl.ds(sid * rps, tail)],
                                hp_sh.at[pl.ds(sid * rps, tail)])

        def serve(idx_hbm, out_hbm):
            pltpu.sync_copy(idx_hbm.at[pl.ds(base, epc)], idx_v)

            def gather(ko, buf, sg):
                pltpu.async_copy(hp_sh.at[idx_v.at[pl.ds(ko * chunk, chunk)]],
                                 buf, sg)

            def gather_wait(ko, buf, sg):
                pltpu.make_async_copy(
                    hp_sh.at[idx_v.at[pl.ds(ko * chunk, chunk)]], buf, sg
                ).wait()

            def wb(ko, buf, sw):
                pltpu.async_copy(buf, out_hbm.at[pl.ds(base + ko * chunk, chunk)],
                                 sw)

            def wb_wait(ko, buf, sw):
                pltpu.make_async_copy(
                    buf, out_hbm.at[pl.ds(base + ko * chunk, chunk)], sw
                ).wait()

            for j in range(min(nb, nch)):
                gather(j, bufs[j], sgs[j])

            nch_main = nch - (nch % nb)

            @pl.loop(0, nch_main, step=nb)
            def _(k0):
                for j in range(nb):
                    gather_wait(k0 + j, bufs[j], sgs[j])
                    wb(k0 + j, bufs[j], sws[j])
                for j in range(nb):
                    wb_wait(k0 + j, bufs[j], sws[j])

                    @pl.when(k0 + j + nb < nch)
                    def _(j=j, k0=k0):
                        gather(k0 + j + nb, bufs[j], sgs[j])

            for c in range(nch_main, nch):
                gather_wait(c, bufs[c % nb], sgs[c % nb])
                wb(c, bufs[c % nb], sws[c % nb])
                wb_wait(c, bufs[c % nb], sws[c % nb])

        @pl.when(cid == 0)
        def _():
            load_hp(hp_s_hbm)

        @pl.when(cid == 1)
        def _():
            load_hp(hp_t_hbm)

        plsc.subcore_barrier()

        @pl.when(cid == 0)
        def _():
            serve(row_hbm, s_hbm)

        @pl.when(cid == 1)
        def _():
            serve(col_hbm, t_hbm)

    return k(hp_s, hp_t, row, col)


def _edge_body(s_ref, t_ref, cd_ref, w1c_ref, b1_ref, w2t_ref, b2_ref, m_ref):
    cd = cd_ref[...]
    radial = jnp.sum(cd * cd, axis=1, keepdims=True)
    x = s_ref[...] + t_ref[...] + radial * w1c_ref[...] + b1_ref[...]
    x = _silu(x)
    y = jnp.dot(x, w2t_ref[...], preferred_element_type=jnp.float32) + b2_ref[...]
    m_ref[...] = _silu(y)


def _sc_scatter(m, row, zeros, n, *, chunk):
    e, d = m.shape
    epw = e // _NW
    rps = -(-n // _NS)
    rps += (-rps) % 8
    tail = n - (_NS - 1) * rps
    assert tail > 0 and rps % 8 == 0
    mesh = plsc.VectorSubcoreMesh(core_axis_name="c", subcore_axis_name="s")

    @functools.partial(
        pl.kernel,
        out_type=jax.ShapeDtypeStruct((_NC, n, d), jnp.float32),
        mesh=mesh,
        scratch_types=[pltpu.VMEM((chunk,), jnp.int32)] * _SNB
                      + [pltpu.VMEM((chunk, d), jnp.float32)] * _SNB
                      + [pltpu.VMEM_SHARED((n, d), jnp.float32)]
                      + [pltpu.SemaphoreType.DMA] * (2 * _SNB),
    )
    def k(m_hbm, row_hbm, z_hbm, out_hbm, *rest):
        ivs = rest[:_SNB]
        mvs = rest[_SNB:2 * _SNB]
        agg_sh = rest[2 * _SNB]
        sis = rest[2 * _SNB + 1:3 * _SNB + 1]
        sms = rest[3 * _SNB + 1:]
        cid = lax.axis_index("c")
        sid = lax.axis_index("s")
        wid = sid * _NC + cid
        base = wid * epw
        nch = epw // chunk

        @pl.when(sid < _NS - 1)
        def _():
            pltpu.sync_copy(z_hbm.at[pl.ds(sid * rps, rps)],
                            agg_sh.at[pl.ds(sid * rps, rps)])

        @pl.when(sid == _NS - 1)
        def _():
            pltpu.sync_copy(z_hbm.at[pl.ds(sid * rps, tail)],
                            agg_sh.at[pl.ds(sid * rps, tail)])

        plsc.subcore_barrier()

        def prefetch(ko, iv, mv, si, sm):
            off = base + ko * chunk
            pltpu.async_copy(row_hbm.at[pl.ds(off, chunk)], iv, si)
            pltpu.async_copy(m_hbm.at[pl.ds(off, chunk)], mv, sm)

        def scatter(ko, iv, mv, si, sm):
            off = base + ko * chunk
            pltpu.make_async_copy(row_hbm.at[pl.ds(off, chunk)], iv, si).wait()
            pltpu.make_async_copy(m_hbm.at[pl.ds(off, chunk)], mv, sm).wait()
            pltpu.sync_copy(mv, agg_sh.at[iv], add=True)

        for j in range(min(_SNB, nch)):
            prefetch(j, ivs[j], mvs[j], sis[j], sms[j])

        nch_main = nch - (nch % _SNB)

        @pl.loop(0, nch_main, step=_SNB)
        def _(k0):
            for j in range(_SNB):
                scatter(k0 + j, ivs[j], mvs[j], sis[j], sms[j])

                @pl.when(k0 + j + _SNB < nch)
                def _(j=j, k0=k0):
                    prefetch(k0 + j + _SNB, ivs[j], mvs[j], sis[j], sms[j])

        for c in range(nch_main, nch):
            scatter(c, ivs[c % _SNB], mvs[c % _SNB], sis[c % _SNB],
                    sms[c % _SNB])

        plsc.subcore_barrier()

        @pl.when(sid < _NS - 1)
        def _():
            pltpu.sync_copy(agg_sh.at[pl.ds(sid * rps, rps)],
                            out_hbm.at[cid].at[pl.ds(sid * rps, rps)])

        @pl.when(sid == _NS - 1)
        def _():
            pltpu.sync_copy(agg_sh.at[pl.ds(sid * rps, tail)],
                            out_hbm.at[cid].at[pl.ds(sid * rps, tail)])

    return k(m, row, zeros)


def _node_body(h_ref, agg_a_ref, agg_b_ref, w3at_ref, w3bt_ref, b3_ref,
               w4t_ref, b4_ref, o_ref):
    agg = (agg_a_ref[0] + agg_a_ref[1]) + (agg_b_ref[0] + agg_b_ref[1])
    x = (jnp.dot(h_ref[...], w3at_ref[...], preferred_element_type=jnp.float32)
         + jnp.dot(agg, w3bt_ref[...], preferred_element_type=jnp.float32)
         + b3_ref[...])
    x = _silu(x)
    o_ref[...] = jnp.dot(x, w4t_ref[...], preferred_element_type=jnp.float32) + b4_ref[...]


def kernel(h, edges_index, coord_diff, w1, b1, w2, b2, w3, b3, w4, b4):
    n, d = h.shape
    e = edges_index.shape[1]
    hd = w1.shape[0]

    ei = edges_index.astype(jnp.int32)
    row = ei[0]
    col = ei[1]

    w1at = w1[:, :d].T
    w1bt = w1[:, d:2 * d].T
    w1c = w1[:, 2 * d].reshape(1, hd)
    b1r = b1.reshape(1, hd)
    w2t = w2.T
    b2r = b2.reshape(1, hd)
    w3at = w3[:, :d].T
    w3bt = w3[:, d:].T
    b3r = b3.reshape(1, hd)
    w4t = w4.T
    b4r = b4.reshape(1, d)

    nblk = 2000
    hp_s, hp_t = pl.pallas_call(
        _proj_body,
        grid=(n // nblk,),
        in_specs=[pl.BlockSpec((nblk, d), lambda i: (i, 0)),
                  pl.BlockSpec((d, hd), lambda i: (0, 0)),
                  pl.BlockSpec((d, hd), lambda i: (0, 0))],
        out_specs=[pl.BlockSpec((nblk, hd), lambda i: (i, 0)),
                   pl.BlockSpec((nblk, hd), lambda i: (i, 0))],
        out_shape=[jax.ShapeDtypeStruct((n, hd), jnp.float32),
                   jax.ShapeDtypeStruct((n, hd), jnp.float32)],
    )(h, w1at, w1bt)

    zeros = jnp.zeros((n, hd), jnp.float32)
    eblk = 2000

    def half(row_h, col_h, cd_h):
        eh = row_h.shape[0]
        s_g, t_g = _sc_gather(hp_s, hp_t, row_h, col_h, chunk=80)
        m = pl.pallas_call(
            _edge_body,
            grid=(eh // eblk,),
            in_specs=[pl.BlockSpec((eblk, hd), lambda i: (i, 0)),
                      pl.BlockSpec((eblk, hd), lambda i: (i, 0)),
                      pl.BlockSpec((eblk, 3), lambda i: (i, 0)),
                      pl.BlockSpec((1, hd), lambda i: (0, 0)),
                      pl.BlockSpec((1, hd), lambda i: (0, 0)),
                      pl.BlockSpec((hd, hd), lambda i: (0, 0)),
                      pl.BlockSpec((1, hd), lambda i: (0, 0))],
            out_specs=pl.BlockSpec((eblk, hd), lambda i: (i, 0)),
            out_shape=jax.ShapeDtypeStruct((eh, hd), jnp.float32),
        )(s_g, t_g, cd_h, w1c, b1r, w2t, b2r)
        return _sc_scatter(m, row_h, zeros, n, chunk=40)

    e2 = e // 2
    agg_a = half(row[:e2], col[:e2], coord_diff[:e2])
    agg_b = half(row[e2:], col[e2:], coord_diff[e2:])

    out = pl.pallas_call(
        _node_body,
        grid=(n // nblk,),
        in_specs=[pl.BlockSpec((nblk, d), lambda i: (i, 0)),
                  pl.BlockSpec((_NC, nblk, hd), lambda i: (0, i, 0)),
                  pl.BlockSpec((_NC, nblk, hd), lambda i: (0, i, 0)),
                  pl.BlockSpec((d, hd), lambda i: (0, 0)),
                  pl.BlockSpec((hd, hd), lambda i: (0, 0)),
                  pl.BlockSpec((1, hd), lambda i: (0, 0)),
                  pl.BlockSpec((hd, d), lambda i: (0, 0)),
                  pl.BlockSpec((1, d), lambda i: (0, 0))],
        out_specs=pl.BlockSpec((nblk, d), lambda i: (i, 0)),
        out_shape=jax.ShapeDtypeStruct((n, d), jnp.float32),
    )(h, agg_a, agg_b, w3at, w3bt, b3r, w4t, b4r)
    return out


# speedup vs baseline: 5.5116x; 1.0008x over previous
import functools

import jax
import jax.numpy as jnp
from jax import lax
from jax.experimental import pallas as pl
from jax.experimental.pallas import tpu as pltpu
from jax.experimental.pallas import tpu_sc as plsc

_NC = 2
_NS = 16
_NW = _NC * _NS
_SNB = 8


def _silu(x):
    return x * jax.nn.sigmoid(x)


def _proj_body(h_ref, w1at_ref, w1bt_ref, s_ref, t_ref):
    hb = h_ref[...]
    s_ref[...] = jnp.dot(hb, w1at_ref[...], preferred_element_type=jnp.float32)
    t_ref[...] = jnp.dot(hb, w1bt_ref[...], preferred_element_type=jnp.float32)


def _sc_gather(hp_s, hp_t, row, col, *, chunk):
    e = row.shape[0]
    n, d = hp_s.shape
    epc = e // _NS
    nch = epc // chunk
    assert nch * chunk == epc
    rps = -(-n // _NS)
    rps += (-rps) % 8
    tail = n - (_NS - 1) * rps
    assert tail > 0 and rps % 8 == 0
    mesh = plsc.VectorSubcoreMesh(core_axis_name="c", subcore_axis_name="s")

    nb = 4

    @functools.partial(
        pl.kernel,
        out_type=[jax.ShapeDtypeStruct((e, d), jnp.float32),
                  jax.ShapeDtypeStruct((e, d), jnp.float32)],
        mesh=mesh,
        scratch_types=[pltpu.VMEM((epc,), jnp.int32)]
                      + [pltpu.VMEM((chunk, d), jnp.float32)] * nb
                      + [pltpu.VMEM_SHARED((n, d), jnp.float32)]
                      + [pltpu.SemaphoreType.DMA] * (2 * nb),
    )
    def k(hp_s_hbm, hp_t_hbm, row_hbm, col_hbm, s_hbm, t_hbm,
          idx_v, *rest):
        bufs = rest[:nb]
        hp_sh = rest[nb]
        sgs = rest[nb + 1:nb + 1 + nb]
        sws = rest[nb + 1 + nb:]
        cid = lax.axis_index("c")
        sid = lax.axis_index("s")
        base = sid * epc

        def load_hp(src_hbm):
            @pl.when(sid < _NS - 1)
            def _():
                pltpu.sync_copy(src_hbm.at[pl.ds(sid * rps, rps)],
                                hp_sh.at[pl.ds(sid * rps, rps)])

            @pl.when(sid == _NS - 1)
            def _():
                pltpu.sync_copy(src_hbm.at[pl.ds(sid * rps, tail)],
                                hp_sh.at[pl.ds(sid * rps, tail)])

        def serve(idx_hbm, out_hbm):
            pltpu.sync_copy(idx_hbm.at[pl.ds(base, epc)], idx_v)

            def gather(ko, buf, sg):
                pltpu.async_copy(hp_sh.at[idx_v.at[pl.ds(ko * chunk, chunk)]],
                                 buf, sg)

            def gather_wait(ko, buf, sg):
                pltpu.make_async_copy(
                    hp_sh.at[idx_v.at[pl.ds(ko * chunk, chunk)]], buf, sg
                ).wait()

            def wb(ko, buf, sw):
                pltpu.async_copy(buf, out_hbm.at[pl.ds(base + ko * chunk, chunk)],
                                 sw)

            def wb_wait(ko, buf, sw):
                pltpu.make_async_copy(
                    buf, out_hbm.at[pl.ds(base + ko * chunk, chunk)], sw
                ).wait()

            for j in range(min(nb, nch)):
                gather(j, bufs[j], sgs[j])

            nch_main = nch - (nch % nb)

            @pl.loop(0, nch_main, step=nb)
            def _(k0):
                for j in range(nb):
                    gather_wait(k0 + j, bufs[j], sgs[j])
                    wb(k0 + j, bufs[j], sws[j])
                for j in range(nb):
                    wb_wait(k0 + j, bufs[j], sws[j])

                    @pl.when(k0 + j + nb < nch)
                    def _(j=j, k0=k0):
                        gather(k0 + j + nb, bufs[j], sgs[j])

            for c in range(nch_main, nch):
                gather_wait(c, bufs[c % nb], sgs[c % nb])
                wb(c, bufs[c % nb], sws[c % nb])
                wb_wait(c, bufs[c % nb], sws[c % nb])

        @pl.when(cid == 0)
        def _():
            load_hp(hp_s_hbm)

        @pl.when(cid == 1)
        def _():
            load_hp(hp_t_hbm)

        plsc.subcore_barrier()

        @pl.when(cid == 0)
        def _():
            serve(row_hbm, s_hbm)

        @pl.when(cid == 1)
        def _():
            serve(col_hbm, t_hbm)

    return k(hp_s, hp_t, row, col)


def _edge_body(s_ref, t_ref, cd_ref, w1c_ref, b1_ref, w2t_ref, b2_ref, m_ref):
    cd = cd_ref[...]
    radial = jnp.sum(cd * cd, axis=1, keepdims=True)
    x = s_ref[...] + t_ref[...] + radial * w1c_ref[...] + b1_ref[...]
    x = _silu(x)
    y = jnp.dot(x, w2t_ref[...], preferred_element_type=jnp.float32) + b2_ref[...]
    m_ref[...] = _silu(y)


def _sc_scatter(m, row, zeros, n, *, chunk):
    e, d = m.shape
    epw = e // _NW
    rps = -(-n // _NS)
    rps += (-rps) % 8
    tail = n - (_NS - 1) * rps
    assert tail > 0 and rps % 8 == 0
    mesh = plsc.VectorSubcoreMesh(core_axis_name="c", subcore_axis_name="s")

    @functools.partial(
        pl.kernel,
        out_type=jax.ShapeDtypeStruct((_NC, n, d), jnp.float32),
        mesh=mesh,
        scratch_types=[pltpu.VMEM((chunk,), jnp.int32)] * _SNB
                      + [pltpu.VMEM((chunk, d), jnp.float32)] * _SNB
                      + [pltpu.VMEM_SHARED((n, d), jnp.float32)]
                      + [pltpu.SemaphoreType.DMA] * (2 * _SNB),
    )
    def k(m_hbm, row_hbm, z_hbm, out_hbm, *rest):
        ivs = rest[:_SNB]
        mvs = rest[_SNB:2 * _SNB]
        agg_sh = rest[2 * _SNB]
        sis = rest[2 * _SNB + 1:3 * _SNB + 1]
        sms = rest[3 * _SNB + 1:]
        cid = lax.axis_index("c")
        sid = lax.axis_index("s")
        wid = sid * _NC + cid
        base = wid * epw
        nch = epw // chunk

        @pl.when(sid < _NS - 1)
        def _():
            pltpu.sync_copy(z_hbm.at[pl.ds(sid * rps, rps)],
                            agg_sh.at[pl.ds(sid * rps, rps)])

        @pl.when(sid == _NS - 1)
        def _():
            pltpu.sync_copy(z_hbm.at[pl.ds(sid * rps, tail)],
                            agg_sh.at[pl.ds(sid * rps, tail)])

        plsc.subcore_barrier()

        def prefetch(ko, iv, mv, si, sm):
            off = base + ko * chunk
            pltpu.async_copy(row_hbm.at[pl.ds(off, chunk)], iv, si)
            pltpu.async_copy(m_hbm.at[pl.ds(off, chunk)], mv, sm)

        def scatter(ko, iv, mv, si, sm):
            off = base + ko * chunk
            pltpu.make_async_copy(row_hbm.at[pl.ds(off, chunk)], iv, si).wait()
            pltpu.make_async_copy(m_hbm.at[pl.ds(off, chunk)], mv, sm).wait()
            pltpu.sync_copy(mv, agg_sh.at[iv], add=True)

        for j in range(min(_SNB, nch)):
            prefetch(j, ivs[j], mvs[j], sis[j], sms[j])

        nch_main = nch - (nch % _SNB)

        @pl.loop(0, nch_main, step=_SNB)
        def _(k0):
            for j in range(_SNB):
                scatter(k0 + j, ivs[j], mvs[j], sis[j], sms[j])

                @pl.when(k0 + j + _SNB < nch)
                def _(j=j, k0=k0):
                    prefetch(k0 + j + _SNB, ivs[j], mvs[j], sis[j], sms[j])

        for c in range(nch_main, nch):
            scatter(c, ivs[c % _SNB], mvs[c % _SNB], sis[c % _SNB],
                    sms[c % _SNB])

        plsc.subcore_barrier()

        @pl.when(sid < _NS - 1)
        def _():
            pltpu.sync_copy(agg_sh.at[pl.ds(sid * rps, rps)],
                            out_hbm.at[cid].at[pl.ds(sid * rps, rps)])

        @pl.when(sid == _NS - 1)
        def _():
            pltpu.sync_copy(agg_sh.at[pl.ds(sid * rps, tail)],
                            out_hbm.at[cid].at[pl.ds(sid * rps, tail)])

    return k(m, row, zeros)


def _node_body(h_ref, agg_a_ref, agg_b_ref, w3at_ref, w3bt_ref, b3_ref,
               w4t_ref, b4_ref, o_ref):
    agg = (agg_a_ref[0] + agg_a_ref[1]) + (agg_b_ref[0] + agg_b_ref[1])
    x = (jnp.dot(h_ref[...], w3at_ref[...], preferred_element_type=jnp.float32)
         + jnp.dot(agg, w3bt_ref[...], preferred_element_type=jnp.float32)
         + b3_ref[...])
    x = _silu(x)
    o_ref[...] = jnp.dot(x, w4t_ref[...], preferred_element_type=jnp.float32) + b4_ref[...]


def kernel(h, edges_index, coord_diff, w1, b1, w2, b2, w3, b3, w4, b4):
    n, d = h.shape
    e = edges_index.shape[1]
    hd = w1.shape[0]

    ei = edges_index.astype(jnp.int32)
    row = ei[0]
    col = ei[1]

    w1at = w1[:, :d].T
    w1bt = w1[:, d:2 * d].T
    w1c = w1[:, 2 * d].reshape(1, hd)
    b1r = b1.reshape(1, hd)
    w2t = w2.T
    b2r = b2.reshape(1, hd)
    w3at = w3[:, :d].T
    w3bt = w3[:, d:].T
    b3r = b3.reshape(1, hd)
    w4t = w4.T
    b4r = b4.reshape(1, d)

    nblk = 2000
    hp_s, hp_t = pl.pallas_call(
        _proj_body,
        grid=(n // nblk,),
        in_specs=[pl.BlockSpec((nblk, d), lambda i: (i, 0)),
                  pl.BlockSpec((d, hd), lambda i: (0, 0)),
                  pl.BlockSpec((d, hd), lambda i: (0, 0))],
        out_specs=[pl.BlockSpec((nblk, hd), lambda i: (i, 0)),
                   pl.BlockSpec((nblk, hd), lambda i: (i, 0))],
        out_shape=[jax.ShapeDtypeStruct((n, hd), jnp.float32),
                   jax.ShapeDtypeStruct((n, hd), jnp.float32)],
    )(h, w1at, w1bt)

    zeros = jnp.zeros((n, hd), jnp.float32)
    eblk = 2000

    def half(row_h, col_h, cd_h):
        eh = row_h.shape[0]
        s_g, t_g = _sc_gather(hp_s, hp_t, row_h, col_h, chunk=80)
        m = pl.pallas_call(
            _edge_body,
            grid=(eh // eblk,),
            in_specs=[pl.BlockSpec((eblk, hd), lambda i: (i, 0)),
                      pl.BlockSpec((eblk, hd), lambda i: (i, 0)),
                      pl.BlockSpec((eblk, 3), lambda i: (i, 0)),
                      pl.BlockSpec((1, hd), lambda i: (0, 0)),
                      pl.BlockSpec((1, hd), lambda i: (0, 0)),
                      pl.BlockSpec((hd, hd), lambda i: (0, 0)),
                      pl.BlockSpec((1, hd), lambda i: (0, 0))],
            out_specs=pl.BlockSpec((eblk, hd), lambda i: (i, 0)),
            out_shape=jax.ShapeDtypeStruct((eh, hd), jnp.float32),
        )(s_g, t_g, cd_h, w1c, b1r, w2t, b2r)
        return _sc_scatter(m, row_h, zeros, n, chunk=40)

    e2 = e // 2
    agg_a = half(row[:e2], col[:e2], coord_diff[:e2])
    agg_b = half(row[e2:], col[e2:], coord_diff[e2:])

    out = pl.pallas_call(
        _node_body,
        grid=(n // nblk,),
        in_specs=[pl.BlockSpec((nblk, d), lambda i: (i, 0)),
                  pl.BlockSpec((_NC, nblk, hd), lambda i: (0, i, 0)),
                  pl.BlockSpec((_NC, nblk, hd), lambda i: (0, i, 0)),
                  pl.BlockSpec((d, hd), lambda i: (0, 0)),
                  pl.BlockSpec((hd, hd), lambda i: (0, 0)),
                  pl.BlockSpec((1, hd), lambda i: (0, 0)),
                  pl.BlockSpec((hd, d), lambda i: (0, 0)),
                  pl.BlockSpec((1, d), lambda i: (0, 0))],
        out_specs=pl.BlockSpec((nblk, d), lambda i: (i, 0)),
        out_shape=jax.ShapeDtypeStruct((n, d), jnp.float32),
    )(h, agg_a, agg_b, w3at, w3bt, b3r, w4t, b4r)
    return out
